# Initial kernel scaffold; baseline (speedup 1.0000x reference)
import jax
import jax.numpy as jnp
from jax.experimental import pallas as pl


def kernel(x, edge_index, snorm_n, snorm_e, e, emb, W0, b0, AL0, ALb0, AR0, ARb0, g0, be0, W1, b1, AL1, ALb1, AR1, ARb1, g1, be1, W2, b2, AL2, ALb2, AR2, ARb2, g2, be2, c1w, c1b, c2w, c2b):
    raise NotImplementedError("write your pallas kernel here")


# speedup vs baseline: 31.8327x; 31.8327x over previous
import functools

import jax
import jax.numpy as jnp
from jax import lax
from jax.experimental import pallas as pl
from jax.experimental.pallas import tpu as pltpu
from jax.experimental.pallas import tpu_sc as plsc

N = 10000
NPAD = 10240
E = 320000
NC = 2
NS = 16
NW = NC * NS
EW = E // NW
EC = 80
NCHUNK = EW // EC
EEC = 80
ENCHUNK = EW // EEC
RPS = NPAD // NS

_MESH = plsc.VectorSubcoreMesh(core_axis_name="c", subcore_axis_name="s")


def _zero_rows(buf, rows, width):
    zv = jnp.zeros((16,), jnp.float32)

    def body(e, _):
        for k in range(width // 16):
            buf[e, pl.ds(k * 16, 16)] = zv
        return 0

    lax.fori_loop(0, rows, body, 0)


def _deg_body(dst_hbm, out_hbm, didx, ones, acc, sem):
    c = lax.axis_index("c")
    s = lax.axis_index("s")
    wid = c * NS + s
    _zero_rows(ones, EC, 128)
    r0 = s * RPS
    off = 0
    while off < RPS:
        step = min(EC, RPS - off)
        pltpu.sync_copy(ones.at[pl.ds(0, step)], acc.at[pl.ds(r0 + off, step)])
        off += step
    ov = jnp.ones((16,), jnp.float32)

    def fill(e, _):
        ones[e, pl.ds(0, 16)] = ov
        return 0

    lax.fori_loop(0, EC, fill, 0)
    plsc.subcore_barrier()

    def chunk(j, _):
        base = wid * EW + j * EC
        pltpu.sync_copy(dst_hbm.at[pl.ds(base, EC)], didx)
        pltpu.sync_copy(ones, acc.at[didx], add=True)
        return 0

    lax.fori_loop(0, NCHUNK, chunk, 0)
    plsc.subcore_barrier()
    pltpu.sync_copy(acc.at[pl.ds(r0, RPS)], out_hbm.at[c, pl.ds(r0, RPS)])


_deg_call = pl.kernel(
    _deg_body,
    out_type=jax.ShapeDtypeStruct((NC, NPAD, 128), jnp.float32),
    mesh=_MESH,
    scratch_types=[
        pltpu.VMEM((EC,), jnp.int32),
        pltpu.VMEM((EC, 128), jnp.float32),
        pltpu.VMEM_SHARED((NPAD, 128), jnp.float32),
        pltpu.SemaphoreType.DMA,
    ],
)


GC = 200
NGCH = N // GC


def _gather0_body(x_hbm, norm_hbm, a0_hbm, t_hbm,
                  xb, nb, ab, a0_sp, sem_a):
    c = lax.axis_index("c")
    s = lax.axis_index("s")
    wid = c * NS + s

    @pl.when(s == 0)
    def _():
        pltpu.sync_copy(a0_hbm, a0_sp)
    plsc.subcore_barrier()

    def do(cid):
        base = cid * GC
        pltpu.sync_copy(x_hbm.at[pl.ds(base, GC)], xb)
        pltpu.sync_copy(norm_hbm.at[pl.ds(base, GC)], nb)
        pltpu.async_copy(a0_sp.at[xb], ab, sem_a).wait()

        def group(g, _):
            nv = nb[pl.ds(g * 16, 16)]
            for lane in range(16):
                e = g * 16 + lane
                ns = nv[lane]
                for k in range(2):
                    ab[e, pl.ds(k * 16, 16)] = ab[e, pl.ds(k * 16, 16)] * ns
            return 0

        lax.fori_loop(0, GC // 16, group, 0)
        if GC % 16:
            nv = nb[pl.ds(GC - 16, 16)]
            for lane in range(16 - GC % 16, 16):
                e = GC - 16 + lane
                ns = nv[lane]
                for k in range(2):
                    ab[e, pl.ds(k * 16, 16)] = ab[e, pl.ds(k * 16, 16)] * ns
        pltpu.sync_copy(ab, t_hbm.at[pl.ds(base, GC)])

    do(wid)

    @pl.when(wid + NW < NGCH)
    def _():
        do(wid + NW)


_gather0_call = pl.kernel(
    _gather0_body,
    out_type=jax.ShapeDtypeStruct((NPAD, 128), jnp.float32),
    mesh=_MESH,
    scratch_types=[
        pltpu.VMEM((GC,), jnp.int32),
        pltpu.VMEM((GC,), jnp.float32),
        pltpu.VMEM((GC, 128), jnp.float32),
        pltpu.VMEM_SHARED((200, 128), jnp.float32),
        pltpu.SemaphoreType.DMA,
    ],
)


def _edge_body(nl, nf, src_hbm, dst_hbm, t_hbm, out_hbm,
               sidx, didx, srows, drows, msg, acc, sem_s, sem_d):
    c = lax.axis_index("c")
    s = lax.axis_index("s")
    wid = c * NS + s
    r0 = s * RPS

    _zero_rows(msg, EEC, 128)
    for t in range(RPS // EEC):
        pltpu.sync_copy(msg, acc.at[pl.ds(r0 + t * EEC, EEC)])
    plsc.subcore_barrier()

    def chunk(j, _):
        base = wid * EW + j * EEC
        pltpu.sync_copy(src_hbm.at[pl.ds(base, EEC)], sidx)
        pltpu.sync_copy(dst_hbm.at[pl.ds(base, EEC)], didx)
        cp1 = pltpu.async_copy(t_hbm.at[sidx], srows, sem_s)
        cp2 = pltpu.async_copy(t_hbm.at[didx], drows, sem_d)
        cp1.wait()
        cp2.wait()

        def edge(e, _):
            al = srows[e, pl.ds(nf, 16)]
            ar = drows[e, pl.ds(nf + 16, 16)]
            f = 1.0 / (1.0 + jnp.exp(-6.0 * (al + ar)))
            fev = [srows[e, pl.ds(k * 16, 16)] for k in range(nf // 16)]
            for li in range(nl):
                fs = f[li]
                for k in range(nf // 16):
                    msg[e, pl.ds(li * nf + k * 16, 16)] = fev[k] * fs
            return 0

        lax.fori_loop(0, EEC, edge, 0)
        pltpu.sync_copy(msg, acc.at[didx], add=True)
        return 0

    lax.fori_loop(0, ENCHUNK, chunk, 0)
    plsc.subcore_barrier()
    pltpu.sync_copy(acc.at[pl.ds(r0, RPS)], out_hbm.at[c, pl.ds(r0, RPS)])


def _make_edge_call(nl, nf):
    return pl.kernel(
        functools.partial(_edge_body, nl, nf),
        out_type=jax.ShapeDtypeStruct((NC, NPAD, 128), jnp.float32),
        mesh=_MESH,
        scratch_types=[
            pltpu.VMEM((EEC,), jnp.int32),
            pltpu.VMEM((EEC,), jnp.int32),
            pltpu.VMEM((EEC, 128), jnp.float32),
            pltpu.VMEM((EEC, 128), jnp.float32),
            pltpu.VMEM((EEC, 128), jnp.float32),
            pltpu.VMEM_SHARED((NPAD, 128), jnp.float32),
            pltpu.SemaphoreType.DMA,
            pltpu.SemaphoreType.DMA,
        ],
    )


_edge_call_0 = _make_edge_call(4, 32)
_edge_call_12 = _make_edge_call(2, 64)


def _tc_prep_body(degp, emb, w0, b0, al0, alb0, ar0, arb0,
                  norm_o, a0_o):
    dv = degp[...]
    deg = dv[0, :N, 0:1] + dv[1, :N, 0:1]
    norm_o[...] = lax.rsqrt(jnp.maximum(deg, 1.0))
    h0 = jnp.dot(emb[...], w0[...]) + b0[...][None, :]
    alv = lax.dot_general(h0, al0[...], (((1,), (1,)), ((), ()))) + alb0[...][None, :]
    arv = lax.dot_general(h0, ar0[...], (((1,), (1,)), ((), ()))) + arb0[...][None, :]
    z12 = jnp.zeros((200, 12), jnp.float32)
    z64 = jnp.zeros((200, 64), jnp.float32)
    a0_o[...] = jnp.concatenate([h0, alv, z12, arv, z12, z64], axis=1)


def _tc_prep(degp, emb, w0, b0, al0, alb0, ar0, arb0):
    return pl.pallas_call(
        _tc_prep_body,
        out_shape=(
            jax.ShapeDtypeStruct((N, 1), jnp.float32),
            jax.ShapeDtypeStruct((200, 128), jnp.float32),
        ),
    )(degp, emb, w0, b0, al0, alb0, ar0, arb0)


def _bn_lrelu(p, snorm, g, be):
    feat = (p[0, :N] + p[1, :N]) * snorm
    mu = jnp.mean(feat, axis=0, keepdims=True)
    d = feat - mu
    var = jnp.mean(d * d, axis=0, keepdims=True)
    feat = (feat - mu) * lax.rsqrt(var + 1e-5) * g[None, :] + be[None, :]
    return jnp.where(feat >= 0, feat, 0.2 * feat)


def _tc_mid_body(p, snorm, g, be, norm, w, b, alw, alb, arw, arb, t_o):
    feat = _bn_lrelu(p[...], snorm[...], g[...], be[...])
    hidden = jnp.dot(feat, w[...]) + b[...][None, :]
    alv = lax.dot_general(hidden, alw[...], (((1,), (1,)), ((), ()))) + alb[...][None, :]
    arv = lax.dot_general(hidden, arw[...], (((1,), (1,)), ((), ()))) + arb[...][None, :]
    fe = hidden * norm[...]
    zr = jnp.zeros((N, 14), jnp.float32)
    z32 = jnp.zeros((N, 32), jnp.float32)
    zpad = jnp.zeros((NPAD - N, 128), jnp.float32)
    t = jnp.concatenate([fe, alv, zr, arv, zr, z32], axis=1)
    t_o[...] = jnp.concatenate([t, zpad], axis=0)


def _tc_mid(p, snorm, g, be, norm, w, b, alw, alb, arw, arb):
    return pl.pallas_call(
        _tc_mid_body,
        out_shape=jax.ShapeDtypeStruct((NPAD, 128), jnp.float32),
    )(p, snorm, g, be, norm, w, b, alw, alb, arw, arb)


def _tc_final_body(p, snorm, g, be, c1w, c1b, c2w, c2b, out_o):
    feat = _bn_lrelu(p[...], snorm[...], g[...], be[...])
    h = jnp.maximum(feat, 0.0)
    h = jnp.maximum(jnp.dot(h, c1w[...]) + c1b[...][None, :], 0.0)
    out_o[...] = jnp.dot(h, c2w[...]) + c2b[...][None, :]


def _tc_final(p, snorm, g, be, c1w, c1b, c2w, c2b):
    return pl.pallas_call(
        _tc_final_body,
        out_shape=jax.ShapeDtypeStruct((N, 2), jnp.float32),
    )(p, snorm, g, be, c1w, c1b, c2w, c2b)


def kernel(x, edge_index, snorm_n, snorm_e, e, emb,
           W0, b0, AL0, ALb0, AR0, ARb0, g0, be0,
           W1, b1, AL1, ALb1, AR1, ARb1, g1, be1,
           W2, b2, AL2, ALb2, AR2, ARb2, g2, be2,
           c1w, c1b, c2w, c2b):
    src = edge_index[0].astype(jnp.int32)
    dst = edge_index[1].astype(jnp.int32)
    xi = x.astype(jnp.int32)

    degp = _deg_call(dst)
    norm, a0t = _tc_prep(degp, emb, W0, b0, AL0, ALb0, AR0, ARb0)
    t0 = _gather0_call(xi, norm[:, 0], a0t)
    p0 = _edge_call_0(src, dst, t0)
    t1 = _tc_mid(p0, snorm_n, g0, be0, norm, W1, b1, AL1, ALb1, AR1, ARb1)
    p1 = _edge_call_12(src, dst, t1)
    t2 = _tc_mid(p1, snorm_n, g1, be1, norm, W2, b2, AL2, ALb2, AR2, ARb2)
    p2 = _edge_call_12(src, dst, t2)
    return _tc_final(p2, snorm_n, g2, be2, c1w, c1b, c2w, c2b)
